# baseline (speedup 1.0000x reference)
import functools

import numpy as np
import jax
import jax.numpy as jnp
from jax import lax
from jax.experimental import pallas as pl

_SAMPLE_N = 4096


def _threefry2x32_bits(hi, lo):
    ks0 = np.uint32(0)
    ks1 = np.uint32(42)
    ks2 = np.uint32(0 ^ 42 ^ 0x1BD11BDA)
    rot_a = (13, 15, 26, 6)
    rot_b = (17, 29, 16, 24)
    sched = (
        (rot_a, ks1, ks2, 1),
        (rot_b, ks2, ks0, 2),
        (rot_a, ks0, ks1, 3),
        (rot_b, ks1, ks2, 4),
        (rot_a, ks2, ks0, 5),
    )
    x0 = hi + ks0
    x1 = lo + ks1
    for rots, a0, a1, c in sched:
        for r in rots:
            x0 = x0 + x1
            x1 = (x1 << np.uint32(r)) | (x1 >> np.uint32(32 - r))
            x1 = x0 ^ x1
        x0 = x0 + a0
        x1 = x1 + a1 + np.uint32(c)
    return x0 ^ x1


def _recip_kernel(p_ref, invp_ref):
    invp_ref[...] = np.float32(1.0) / p_ref[...]


def _sample_kernel(invp_ref, out_ref, *, B, NV, NC, Vm):
    i = pl.program_id(0)
    lane = lax.broadcasted_iota(jnp.int32, (8, NV), 1)
    rows = lax.broadcasted_iota(jnp.int32, (8, 1), 0)
    tiny = np.float32(np.finfo(np.float32).tiny)
    scale = np.float32(1.0) - tiny
    big_idx = np.int32(2**31 - 1)

    wins = []
    for bg in range(B // 8):
        t = i * np.int32(B) + np.int32(bg * 8) + rows
        tvm = (t * np.int32(Vm)).astype(jnp.uint32)
        hi_base = tvm >> np.uint32(27)
        lo_base = tvm << np.uint32(5)

        def body(c, carry, bg=bg, hi_base=hi_base, lo_base=lo_base):
            score0, idx0 = carry
            v = c * NV + lane
            vu = v.astype(jnp.uint32)
            lo = lo_base + vu
            hi = hi_base + (lo < lo_base).astype(jnp.uint32)
            bits = _threefry2x32_bits(hi, lo)
            fb = (bits >> np.uint32(9)) | np.uint32(0x3F800000)
            u = lax.bitcast_convert_type(fb, jnp.float32) - np.float32(1.0)
            u = jnp.maximum(tiny, u * scale + tiny)
            e = -jnp.log(u)
            s = e * invp_ref[c, bg * 8:(bg + 1) * 8, :]
            pred = s < score0
            return (jnp.where(pred, s, score0), jnp.where(pred, v, idx0))

        score, idx = lax.fori_loop(
            0, NC, body,
            (jnp.full((8, NV), jnp.inf, jnp.float32),
             jnp.zeros((8, NV), jnp.int32)))
        m = jnp.min(score, axis=1, keepdims=True)
        cand = jnp.where(score == m, idx, big_idx)
        wins.append(jnp.min(cand, axis=1))
    out_ref[0, 0, :] = jnp.concatenate(wins)


def _build(probs, n, nv, interpret=False):
    B, V = probs.shape
    assert V % 32 == 0 or True
    vm, rem = divmod(V, 32)
    if rem:
        raise ValueError("V must be a multiple of 32")
    nc = -(-V // nv)
    pv = nc * nv
    probs_pad = jnp.pad(probs, ((0, 0), (0, pv - V)))
    probs3 = probs_pad.reshape(B, nc, nv).transpose(1, 0, 2)
    invp3 = pl.pallas_call(
        _recip_kernel,
        out_shape=jax.ShapeDtypeStruct((nc, B, nv), jnp.float32),
        interpret=interpret,
    )(probs3)
    out3 = pl.pallas_call(
        functools.partial(_sample_kernel, B=B, NV=nv, NC=nc, Vm=vm),
        grid=(n,),
        in_specs=[pl.BlockSpec((nc, B, nv), lambda i: (0, 0, 0))],
        out_specs=pl.BlockSpec((1, 1, B), lambda i: (i, 0, 0)),
        out_shape=jax.ShapeDtypeStruct((n, 1, B), jnp.int32),
        interpret=interpret,
    )(invp3)
    return out3.reshape(n, B)


def kernel(probs):
    return _build(probs, _SAMPLE_N, 512)


# speedup vs baseline: 1.5503x; 1.5503x over previous
import functools

import numpy as np
import jax
import jax.numpy as jnp
from jax import lax
from jax.experimental import pallas as pl

_SAMPLE_N = 4096


def _threefry2x32_bits(hi, lo):
    ks0 = np.uint32(0)
    ks1 = np.uint32(42)
    ks2 = np.uint32(0 ^ 42 ^ 0x1BD11BDA)
    rot_a = (13, 15, 26, 6)
    rot_b = (17, 29, 16, 24)
    sched = (
        (rot_a, ks1, ks2, 1),
        (rot_b, ks2, ks0, 2),
        (rot_a, ks0, ks1, 3),
        (rot_b, ks1, ks2, 4),
        (rot_a, ks2, ks0, 5),
    )
    x0 = hi + ks0
    x1 = lo + ks1
    for rots, a0, a1, c in sched:
        for r in rots:
            x0 = x0 + x1
            x1 = (x1 << np.uint32(r)) | (x1 >> np.uint32(32 - r))
            x1 = x0 ^ x1
        x0 = x0 + a0
        x1 = x1 + a1 + np.uint32(c)
    return x0 ^ x1


def _recip_kernel(p_ref, invp_ref):
    invp_ref[...] = np.float32(1.0) / p_ref[...]


def _sample_kernel(invp_ref, out_ref, *, B, NV, NC, Vm):
    i = pl.program_id(0)
    lane = lax.broadcasted_iota(jnp.uint32, (8, NV), 1)
    rows = lax.broadcasted_iota(jnp.int32, (8, 1), 0)
    tiny = np.float32(np.finfo(np.float32).tiny)
    big_idx = np.int32(2**31 - 1)

    wins = []
    for bg in range(B // 8):
        t = i * np.int32(B) + np.int32(bg * 8) + rows
        tvm = (t * np.int32(Vm)).astype(jnp.uint32)
        hi_base0 = tvm >> np.uint32(27)
        lo_base0 = tvm << np.uint32(5)
        lo_base = lo_base0 + lane
        hi_base = hi_base0 + (lo_base < lo_base0).astype(jnp.uint32)

        def body(c, carry, bg=bg, hi_base=hi_base, lo_base=lo_base):
            score0, cwin0 = carry
            step = (c * NV).astype(jnp.uint32)
            lo = lo_base + step
            hi = hi_base + (lo < lo_base).astype(jnp.uint32)
            bits = _threefry2x32_bits(hi, lo)
            fb = (bits >> np.uint32(9)) | np.uint32(0x3F800000)
            u = lax.bitcast_convert_type(fb, jnp.float32) - np.float32(1.0)
            u = jnp.maximum(u, tiny)
            e = -jnp.log(u)
            s = e * invp_ref[c, bg * 8:(bg + 1) * 8, :]
            pred = s < score0
            return (jnp.where(pred, s, score0), jnp.where(pred, c, cwin0))

        score, cwin = lax.fori_loop(
            0, NC, body,
            (jnp.full((8, NV), jnp.inf, jnp.float32),
             jnp.zeros((8, NV), jnp.int32)))
        idx = cwin * NV + lane.astype(jnp.int32)
        m = jnp.min(score, axis=1, keepdims=True)
        cand = jnp.where(score == m, idx, big_idx)
        wins.append(jnp.min(cand, axis=1))
    out_ref[0, 0, :] = jnp.concatenate(wins)


def _build(probs, n, nv, interpret=False):
    B, V = probs.shape
    assert V % 32 == 0 or True
    vm, rem = divmod(V, 32)
    if rem:
        raise ValueError("V must be a multiple of 32")
    nc = -(-V // nv)
    pv = nc * nv
    probs_pad = jnp.pad(probs, ((0, 0), (0, pv - V)))
    probs3 = probs_pad.reshape(B, nc, nv).transpose(1, 0, 2)
    invp3 = pl.pallas_call(
        _recip_kernel,
        out_shape=jax.ShapeDtypeStruct((nc, B, nv), jnp.float32),
        interpret=interpret,
    )(probs3)
    out3 = pl.pallas_call(
        functools.partial(_sample_kernel, B=B, NV=nv, NC=nc, Vm=vm),
        grid=(n,),
        in_specs=[pl.BlockSpec((nc, B, nv), lambda i: (0, 0, 0))],
        out_specs=pl.BlockSpec((1, 1, B), lambda i: (i, 0, 0)),
        out_shape=jax.ShapeDtypeStruct((n, 1, B), jnp.int32),
        interpret=interpret,
    )(invp3)
    return out3.reshape(n, B)


def kernel(probs):
    return _build(probs, _SAMPLE_N, 2048)
